# baseline (speedup 1.0000x reference)
import jax
import jax.numpy as jnp
from jax.experimental import pallas as pl
from jax.experimental.pallas import tpu as pltpu

VOCAB = 32768
HIDDEN = 2048
NUM_EXPERTS = 16
TOP_K = 2
TOKENS = 512
EXPERT_VOCAB = VOCAB // NUM_EXPERTS
VSPLIT = 4
EVBLK = EXPERT_VOCAB // VSPLIT


def _moe_head_body(x_ref, w_ref, rw_ref, out_ref, sel_ref):
    e = pl.program_id(0)
    v = pl.program_id(1)

    @pl.when((e == 0) & (v == 0))
    def _routing():
        x = x_ref[...]
        rw = rw_ref[...]
        logits = jnp.dot(x, rw.T, preferred_element_type=jnp.float32)
        m = jnp.max(logits, axis=1, keepdims=True)
        w = jnp.exp(logits - m)
        w = w / jnp.sum(w, axis=1, keepdims=True)
        a1 = jnp.argmax(w, axis=1)
        eids = jax.lax.broadcasted_iota(jnp.int32, (TOKENS, NUM_EXPERTS), 1)
        w2 = jnp.where(eids == a1[:, None], -jnp.inf, w)
        a2 = jnp.argmax(w2, axis=1)
        sel = (eids == a1[:, None]) | (eids == a2[:, None])
        sel_ref[...] = sel.astype(jnp.float32)

    xb = x_ref[...].astype(jnp.bfloat16)
    wb = w_ref[0].astype(jnp.bfloat16)
    acc = jnp.dot(xb, wb.T, preferred_element_type=jnp.float32)
    onehot = (jax.lax.broadcasted_iota(jnp.int32, (1, NUM_EXPERTS), 1) == e)
    selcol = jnp.sum(sel_ref[...] * onehot, axis=1, keepdims=True) > 0.0
    out_ref[...] = jnp.where(selcol, acc, -jnp.inf)


def kernel(hidden_states, expert_weight, router_weight):
    return pl.pallas_call(
        _moe_head_body,
        grid=(NUM_EXPERTS, VSPLIT),
        in_specs=[
            pl.BlockSpec((TOKENS, HIDDEN), lambda e, v: (0, 0)),
            pl.BlockSpec((1, EVBLK, HIDDEN), lambda e, v: (e, v, 0)),
            pl.BlockSpec((NUM_EXPERTS, HIDDEN), lambda e, v: (0, 0)),
        ],
        out_specs=pl.BlockSpec((TOKENS, EVBLK), lambda e, v: (0, e * VSPLIT + v)),
        out_shape=jax.ShapeDtypeStruct((TOKENS, VOCAB), jnp.float32),
        scratch_shapes=[pltpu.VMEM((TOKENS, NUM_EXPERTS), jnp.float32)],
    )(hidden_states, expert_weight, router_weight)


# speedup vs baseline: 1.2314x; 1.2314x over previous
import jax
import jax.numpy as jnp
from jax.experimental import pallas as pl
from jax.experimental.pallas import tpu as pltpu

VOCAB = 32768
HIDDEN = 2048
NUM_EXPERTS = 16
TOP_K = 2
TOKENS = 512
EXPERT_VOCAB = VOCAB // NUM_EXPERTS
VSPLIT = 1
EVBLK = EXPERT_VOCAB // VSPLIT


def _moe_head_body(x_ref, w_ref, rw_ref, out_ref, sel_ref):
    e = pl.program_id(0)
    v = pl.program_id(1)

    @pl.when((e == 0) & (v == 0))
    def _routing():
        x = x_ref[...]
        rw = rw_ref[...]
        logits = jnp.dot(x, rw.T, preferred_element_type=jnp.float32)
        m = jnp.max(logits, axis=1, keepdims=True)
        w = jnp.exp(logits - m)
        w = w / jnp.sum(w, axis=1, keepdims=True)
        a1 = jnp.argmax(w, axis=1)
        eids = jax.lax.broadcasted_iota(jnp.int32, (TOKENS, NUM_EXPERTS), 1)
        w2 = jnp.where(eids == a1[:, None], -jnp.inf, w)
        a2 = jnp.argmax(w2, axis=1)
        sel = (eids == a1[:, None]) | (eids == a2[:, None])
        sel_ref[...] = sel.astype(jnp.float32)

    xb = x_ref[...].astype(jnp.bfloat16)
    wb = w_ref[0].astype(jnp.bfloat16)
    acc = jnp.dot(xb, wb.T, preferred_element_type=jnp.float32)
    onehot = (jax.lax.broadcasted_iota(jnp.int32, (1, NUM_EXPERTS), 1) == e)
    selcol = jnp.sum(sel_ref[...] * onehot, axis=1, keepdims=True) > 0.0
    out_ref[...] = jnp.where(selcol, acc, -jnp.inf)


def kernel(hidden_states, expert_weight, router_weight):
    return pl.pallas_call(
        _moe_head_body,
        grid=(NUM_EXPERTS, VSPLIT),
        in_specs=[
            pl.BlockSpec((TOKENS, HIDDEN), lambda e, v: (0, 0)),
            pl.BlockSpec((1, EVBLK, HIDDEN), lambda e, v: (e, v, 0)),
            pl.BlockSpec((NUM_EXPERTS, HIDDEN), lambda e, v: (0, 0)),
        ],
        out_specs=pl.BlockSpec((TOKENS, EVBLK), lambda e, v: (0, e * VSPLIT + v)),
        out_shape=jax.ShapeDtypeStruct((TOKENS, VOCAB), jnp.float32),
        scratch_shapes=[pltpu.VMEM((TOKENS, NUM_EXPERTS), jnp.float32)],
    )(hidden_states, expert_weight, router_weight)
